# baseline (speedup 1.0000x reference)
import functools

import jax
import jax.numpy as jnp
from jax import lax
from jax.experimental import pallas as pl
from jax.experimental.pallas import tpu as pltpu
from jax.experimental.pallas import tpu_sc as plsc

B = 1000
NPG = 100
N = B * NPG
D = 128
H = 128

GB = 100
R = GB * NPG
GRID = B // GB

_NC, _NS = 2, 16
_NW = _NC * _NS
BP = 1024
BPW = BP // _NW


@functools.cache
def _get_sc_gather():
    mesh = plsc.VectorSubcoreMesh(core_axis_name="c", subcore_axis_name="s")

    @functools.partial(
        pl.kernel,
        mesh=mesh,
        out_type=[
            jax.ShapeDtypeStruct((BP, D), jnp.float32),
            jax.ShapeDtypeStruct((BP, D), jnp.float32),
        ],
        scratch_types=[
            pltpu.VMEM((BPW,), jnp.int32),
            pltpu.VMEM((BPW, D), jnp.float32),
            pltpu.VMEM((BPW, D), jnp.float32),
            pltpu.SemaphoreType.DMA,
            pltpu.SemaphoreType.DMA,
        ],
    )
    def _sc_gather(ti_hbm, tv_hbm, idx_hbm, oi_hbm, ov_hbm,
                   idx_v, ri_v, rv_v, s1, s2):
        wid = lax.axis_index("s") * _NC + lax.axis_index("c")
        base = wid * BPW
        pltpu.sync_copy(idx_hbm.at[pl.ds(base, BPW)], idx_v)
        c1 = pltpu.async_copy(ti_hbm.at[idx_v], ri_v, s1)
        c2 = pltpu.async_copy(tv_hbm.at[idx_v], rv_v, s2)
        c1.wait()
        c2.wait()
        pltpu.sync_copy(ri_v, oi_hbm.at[pl.ds(base, BPW)])
        pltpu.sync_copy(rv_v, ov_hbm.at[pl.ds(base, BPW)])

    return _sc_gather


def _attn_block(xi_ref, xv_ref, fvi_ref, fvv_ref, wut_ref, but_ref, wvt_ref,
                wet_ref, sgr_ref, srg_ref, oi_ref, ov_ref):
    xiT = xi_ref[...].T
    xvT = xv_ref[...].T
    wut = wut_ref[...].astype(jnp.bfloat16)
    uiT = jnp.dot(wut, xiT.astype(jnp.bfloat16),
                  preferred_element_type=jnp.float32
                  ).astype(jnp.bfloat16)
    uvT = jnp.dot(wut, xvT.astype(jnp.bfloat16),
                  preferred_element_type=jnp.float32).astype(jnp.bfloat16)
    wvt = wvt_ref[...].astype(jnp.bfloat16)
    but = but_ref[...].astype(jnp.bfloat16)
    qiT = jnp.dot(wvt, fvi_ref[0].T.astype(jnp.bfloat16),
                  preferred_element_type=jnp.float32
                  ).astype(jnp.bfloat16) + but
    qvT = jnp.dot(wvt, fvv_ref[0].T.astype(jnp.bfloat16),
                  preferred_element_type=jnp.float32).astype(jnp.bfloat16) + but

    smat_gr = sgr_ref[...]
    smat_rg = srg_ref[...]
    wet = wet_ref[...].astype(jnp.bfloat16)

    def one_query(qT, out_ref):
        bigQ = jnp.dot(qT, smat_gr,
                       preferred_element_type=jnp.float32
                       ).astype(jnp.bfloat16)
        ei = jnp.dot(wet, jax.nn.sigmoid(uiT + bigQ),
                     preferred_element_type=jnp.float32)
        ev = jnp.dot(wet, jax.nn.sigmoid(uvT + bigQ),
                     preferred_element_type=jnp.float32)
        m = jnp.max(jnp.maximum(ei, ev))
        wi = jnp.exp(ei - m)
        wv_ = jnp.exp(ev - m)
        denom = jnp.dot(wi + wv_, smat_rg,
                        preferred_element_type=jnp.float32)
        zT = xiT * wi + xvT * wv_
        rstT = jnp.dot(zT, smat_rg,
                       preferred_element_type=jnp.float32)
        out_ref[...] = (rstT / denom)[None]

    one_query(qiT, oi_ref)
    one_query(qvT, ov_ref)


def _tc_call(feat_invar, feat_var, fvi, fvv, wut, but, wvt, wet, sgr, srg):
    full = lambda shape: pl.BlockSpec(shape, lambda i: (0, 0))
    return pl.pallas_call(
        _attn_block,
        grid=(GRID,),
        in_specs=[
            pl.BlockSpec((R, D), lambda i: (i, 0)),
            pl.BlockSpec((R, D), lambda i: (i, 0)),
            pl.BlockSpec((1, GB, D), lambda i: (i, 0, 0)),
            pl.BlockSpec((1, GB, D), lambda i: (i, 0, 0)),
            full((H, D)),
            full((H, 1)),
            full((H, D)),
            full((1, H)),
            full((GB, R)),
            full((R, GB)),
        ],
        out_specs=[
            pl.BlockSpec((1, D, GB), lambda i: (i, 0, 0)),
            pl.BlockSpec((1, D, GB), lambda i: (i, 0, 0)),
        ],
        out_shape=[
            jax.ShapeDtypeStruct((GRID, D, GB), jnp.float32),
            jax.ShapeDtypeStruct((GRID, D, GB), jnp.float32),
        ],
    )(feat_invar, feat_var, fvi, fvv, wut, but, wvt, wet, sgr, srg)


def kernel(feat_invar, feat_var, last_nodes, Wu, bu, Wv, We):
    idx = jnp.pad(last_nodes.astype(jnp.int32), (0, BP - B))
    fvi = feat_invar[idx]
    fvv = feat_var[idx]
    onehot = (jnp.arange(R)[:, None] // NPG) == jnp.arange(GB)[None, :]
    srg = onehot.astype(jnp.float32)
    sgr = onehot.T.astype(jnp.bfloat16)
    fvi = fvi[:B].reshape(GRID, GB, D)
    fvv = fvv[:B].reshape(GRID, GB, D)
    o3i, o3v = _tc_call(feat_invar, feat_var, fvi, fvv,
                        Wu.T, bu.reshape(H, 1), Wv.T, We.reshape(1, H),
                        sgr, srg)
    rst_i = o3i.transpose(0, 2, 1).reshape(B, D)
    rst_v = o3v.transpose(0, 2, 1).reshape(B, D)
    return rst_i[:, None, :], rst_v[:, None, :]


# speedup vs baseline: 1.1163x; 1.1163x over previous
import functools

import jax
import jax.numpy as jnp
from jax import lax
from jax.experimental import pallas as pl
from jax.experimental.pallas import tpu as pltpu
from jax.experimental.pallas import tpu_sc as plsc

B = 1000
NPG = 100
N = B * NPG
D = 128
H = 128

GB = 100
R = GB * NPG
GRID = B // GB

_NC, _NS = 2, 16
_NW = _NC * _NS
BP = 1024
BPW = BP // _NW


@functools.cache
def _get_sc_gather():
    mesh = plsc.VectorSubcoreMesh(core_axis_name="c", subcore_axis_name="s")

    @functools.partial(
        pl.kernel,
        mesh=mesh,
        out_type=[
            jax.ShapeDtypeStruct((BP, D), jnp.float32),
            jax.ShapeDtypeStruct((BP, D), jnp.float32),
        ],
        scratch_types=[
            pltpu.VMEM((BPW,), jnp.int32),
            pltpu.VMEM((BPW, D), jnp.float32),
            pltpu.VMEM((BPW, D), jnp.float32),
            pltpu.SemaphoreType.DMA,
            pltpu.SemaphoreType.DMA,
        ],
    )
    def _sc_gather(ti_hbm, tv_hbm, idx_hbm, oi_hbm, ov_hbm,
                   idx_v, ri_v, rv_v, s1, s2):
        wid = lax.axis_index("s") * _NC + lax.axis_index("c")
        base = wid * BPW
        pltpu.sync_copy(idx_hbm.at[pl.ds(base, BPW)], idx_v)
        c1 = pltpu.async_copy(ti_hbm.at[idx_v], ri_v, s1)
        c2 = pltpu.async_copy(tv_hbm.at[idx_v], rv_v, s2)
        c1.wait()
        c2.wait()
        pltpu.sync_copy(ri_v, oi_hbm.at[pl.ds(base, BPW)])
        pltpu.sync_copy(rv_v, ov_hbm.at[pl.ds(base, BPW)])

    return _sc_gather


def _attn_block(xi_ref, xv_ref, fvi_ref, fvv_ref, wut_ref, but_ref, wvt_ref,
                wet_ref, sgr_ref, srg_ref, oi_ref, ov_ref):
    xiT = xi_ref[...].T
    xvT = xv_ref[...].T
    wut = wut_ref[...].astype(jnp.bfloat16)
    uiT = jnp.dot(wut, xiT.astype(jnp.bfloat16),
                  preferred_element_type=jnp.float32
                  ).astype(jnp.bfloat16)
    uvT = jnp.dot(wut, xvT.astype(jnp.bfloat16),
                  preferred_element_type=jnp.float32).astype(jnp.bfloat16)
    wvt = wvt_ref[...].astype(jnp.bfloat16)
    but = but_ref[...].astype(jnp.bfloat16)
    qiT = jnp.dot(wvt, fvi_ref[0].T.astype(jnp.bfloat16),
                  preferred_element_type=jnp.float32
                  ).astype(jnp.bfloat16) + but
    qvT = jnp.dot(wvt, fvv_ref[0].T.astype(jnp.bfloat16),
                  preferred_element_type=jnp.float32).astype(jnp.bfloat16) + but

    smat_gr = sgr_ref[...]
    smat_rg = srg_ref[...]
    wet = wet_ref[...].astype(jnp.bfloat16)

    def one_query(qT, out_ref):
        bigQ = jnp.dot(qT, smat_gr,
                       preferred_element_type=jnp.float32
                       ).astype(jnp.bfloat16)
        ei = jnp.dot(wet, jax.nn.sigmoid(uiT + bigQ),
                     preferred_element_type=jnp.float32)
        ev = jnp.dot(wet, jax.nn.sigmoid(uvT + bigQ),
                     preferred_element_type=jnp.float32)
        m = jnp.max(jnp.maximum(ei, ev))
        wi = jnp.exp(ei - m)
        wv_ = jnp.exp(ev - m)
        denom = jnp.dot(wi + wv_, smat_rg,
                        preferred_element_type=jnp.float32)
        zT = xiT * wi + xvT * wv_
        rstT = jnp.dot(zT, smat_rg,
                       preferred_element_type=jnp.float32)
        out_ref[...] = (rstT / denom)[None]

    one_query(qiT, oi_ref)
    one_query(qvT, ov_ref)


def _tc_call(feat_invar, feat_var, fvi, fvv, wut, but, wvt, wet, sgr, srg):
    full = lambda shape: pl.BlockSpec(shape, lambda i: (0, 0))
    return pl.pallas_call(
        _attn_block,
        grid=(GRID,),
        in_specs=[
            pl.BlockSpec((R, D), lambda i: (i, 0)),
            pl.BlockSpec((R, D), lambda i: (i, 0)),
            pl.BlockSpec((1, GB, D), lambda i: (i, 0, 0)),
            pl.BlockSpec((1, GB, D), lambda i: (i, 0, 0)),
            full((H, D)),
            full((H, 1)),
            full((H, D)),
            full((1, H)),
            full((GB, R)),
            full((R, GB)),
        ],
        out_specs=[
            pl.BlockSpec((1, D, GB), lambda i: (i, 0, 0)),
            pl.BlockSpec((1, D, GB), lambda i: (i, 0, 0)),
        ],
        out_shape=[
            jax.ShapeDtypeStruct((GRID, D, GB), jnp.float32),
            jax.ShapeDtypeStruct((GRID, D, GB), jnp.float32),
        ],
    )(feat_invar, feat_var, fvi, fvv, wut, but, wvt, wet, sgr, srg)


def kernel(feat_invar, feat_var, last_nodes, Wu, bu, Wv, We):
    idx = jnp.pad(last_nodes.astype(jnp.int32), (0, BP - B))
    fvi, fvv = _get_sc_gather()(feat_invar, feat_var, idx)
    onehot = (jnp.arange(R)[:, None] // NPG) == jnp.arange(GB)[None, :]
    srg = onehot.astype(jnp.float32)
    sgr = onehot.T.astype(jnp.bfloat16)
    fvi = fvi[:B].reshape(GRID, GB, D)
    fvv = fvv[:B].reshape(GRID, GB, D)
    o3i, o3v = _tc_call(feat_invar, feat_var, fvi, fvv,
                        Wu.T, bu.reshape(H, 1), Wv.T, We.reshape(1, H),
                        sgr, srg)
    rst_i = o3i.transpose(0, 2, 1).reshape(B, D)
    rst_v = o3v.transpose(0, 2, 1).reshape(B, D)
    return rst_i[:, None, :], rst_v[:, None, :]


# speedup vs baseline: 1.1541x; 1.0339x over previous
import functools

import jax
import jax.numpy as jnp
import numpy as np
from jax import lax
from jax.experimental import pallas as pl
from jax.experimental.pallas import tpu as pltpu
from jax.experimental.pallas import tpu_sc as plsc

B = 1000
NPG = 100
N = B * NPG
D = 128
H = 128

GB = 100
R = GB * NPG
GRID = B // GB

_NC, _NS = 2, 16
_NW = _NC * _NS
BP = 1024
BPW = BP // _NW


@functools.cache
def _get_sc_gather():
    mesh = plsc.VectorSubcoreMesh(core_axis_name="c", subcore_axis_name="s")

    @functools.partial(
        pl.kernel,
        mesh=mesh,
        out_type=[
            jax.ShapeDtypeStruct((BP, D), jnp.float32),
            jax.ShapeDtypeStruct((BP, D), jnp.float32),
        ],
        scratch_types=[
            pltpu.VMEM((BPW,), jnp.int32),
            pltpu.VMEM((BPW, D), jnp.float32),
            pltpu.VMEM((BPW, D), jnp.float32),
            pltpu.SemaphoreType.DMA,
            pltpu.SemaphoreType.DMA,
        ],
    )
    def _sc_gather(ti_hbm, tv_hbm, idx_hbm, oi_hbm, ov_hbm,
                   idx_v, ri_v, rv_v, s1, s2):
        wid = lax.axis_index("s") * _NC + lax.axis_index("c")
        base = wid * BPW
        pltpu.sync_copy(idx_hbm.at[pl.ds(base, BPW)], idx_v)
        c1 = pltpu.async_copy(ti_hbm.at[idx_v], ri_v, s1)
        c2 = pltpu.async_copy(tv_hbm.at[idx_v], rv_v, s2)
        c1.wait()
        c2.wait()
        pltpu.sync_copy(ri_v, oi_hbm.at[pl.ds(base, BPW)])
        pltpu.sync_copy(rv_v, ov_hbm.at[pl.ds(base, BPW)])

    return _sc_gather


def _attn_block(xi_ref, xv_ref, fvi_ref, fvv_ref, wut_ref, but_ref, wvt_ref,
                wet_ref, sgr_ref, srg_ref, oi_ref, ov_ref):
    xiT = xi_ref[...].T
    xvT = xv_ref[...].T
    wut = wut_ref[...].astype(jnp.bfloat16)
    uiT = jnp.dot(wut, xiT.astype(jnp.bfloat16),
                  preferred_element_type=jnp.float32
                  ).astype(jnp.bfloat16)
    uvT = jnp.dot(wut, xvT.astype(jnp.bfloat16),
                  preferred_element_type=jnp.float32).astype(jnp.bfloat16)
    wvt = wvt_ref[...].astype(jnp.bfloat16)
    but = but_ref[...].astype(jnp.bfloat16)
    qiT = jnp.dot(wvt, fvi_ref[0].T.astype(jnp.bfloat16),
                  preferred_element_type=jnp.float32
                  ).astype(jnp.bfloat16) + but
    qvT = jnp.dot(wvt, fvv_ref[0].T.astype(jnp.bfloat16),
                  preferred_element_type=jnp.float32).astype(jnp.bfloat16) + but

    smat_gr = sgr_ref[...]
    smat_rg = srg_ref[...]
    wet = wet_ref[...].astype(jnp.bfloat16)

    def one_query(qT, out_ref):
        bigQ = jnp.dot(qT, smat_gr,
                       preferred_element_type=jnp.float32
                       ).astype(jnp.bfloat16)
        ei = jnp.dot(wet, jax.nn.sigmoid(uiT + bigQ),
                     preferred_element_type=jnp.float32)
        ev = jnp.dot(wet, jax.nn.sigmoid(uvT + bigQ),
                     preferred_element_type=jnp.float32)
        m = jnp.max(jnp.maximum(ei, ev))
        wi = jnp.exp(ei - m)
        wv_ = jnp.exp(ev - m)
        denom = jnp.dot(wi + wv_, smat_rg,
                        preferred_element_type=jnp.float32)
        zT = xiT * wi + xvT * wv_
        rstT = jnp.dot(zT, smat_rg,
                       preferred_element_type=jnp.float32)
        out_ref[...] = (rstT / denom).T[None]

    one_query(qiT, oi_ref)
    one_query(qvT, ov_ref)


def _tc_call(feat_invar, feat_var, fvi, fvv, wut, but, wvt, wet, sgr, srg):
    full = lambda shape: pl.BlockSpec(shape, lambda i: (0, 0))
    return pl.pallas_call(
        _attn_block,
        grid=(GRID,),
        in_specs=[
            pl.BlockSpec((R, D), lambda i: (i, 0)),
            pl.BlockSpec((R, D), lambda i: (i, 0)),
            pl.BlockSpec((1, GB, D), lambda i: (i, 0, 0)),
            pl.BlockSpec((1, GB, D), lambda i: (i, 0, 0)),
            full((H, D)),
            full((H, 1)),
            full((H, D)),
            full((1, H)),
            full((GB, R)),
            full((R, GB)),
        ],
        out_specs=[
            pl.BlockSpec((1, GB, D), lambda i: (i, 0, 0)),
            pl.BlockSpec((1, GB, D), lambda i: (i, 0, 0)),
        ],
        out_shape=[
            jax.ShapeDtypeStruct((GRID, GB, D), jnp.float32),
            jax.ShapeDtypeStruct((GRID, GB, D), jnp.float32),
        ],
    )(feat_invar, feat_var, fvi, fvv, wut, but, wvt, wet, sgr, srg)


def kernel(feat_invar, feat_var, last_nodes, Wu, bu, Wv, We):
    idx = jnp.pad(last_nodes.astype(jnp.int32), (0, BP - B))
    fvi, fvv = _get_sc_gather()(feat_invar, feat_var, idx)
    onehot = (np.arange(R)[:, None] // NPG) == np.arange(GB)[None, :]
    srg = jnp.asarray(onehot, jnp.float32)
    sgr = jnp.asarray(onehot.T, jnp.bfloat16)
    fvi = fvi[:B].reshape(GRID, GB, D)
    fvv = fvv[:B].reshape(GRID, GB, D)
    o3i, o3v = _tc_call(feat_invar, feat_var, fvi, fvv,
                        Wu.T, bu.reshape(H, 1), Wv.T, We.reshape(1, H),
                        sgr, srg)
    rst_i = o3i.reshape(B, D)
    rst_v = o3v.reshape(B, D)
    return rst_i[:, None, :], rst_v[:, None, :]


# speedup vs baseline: 1.1549x; 1.0007x over previous
import functools

import jax
import jax.numpy as jnp
import numpy as np
from jax import lax
from jax.experimental import pallas as pl
from jax.experimental.pallas import tpu as pltpu
from jax.experimental.pallas import tpu_sc as plsc

B = 1000
NPG = 100
N = B * NPG
D = 128
H = 128

GB = 100
R = GB * NPG
GRID = B // GB

_NC, _NS = 2, 16
_NW = _NC * _NS
BP = 1024
BPW = BP // _NW


@functools.cache
def _get_sc_gather():
    mesh = plsc.VectorSubcoreMesh(core_axis_name="c", subcore_axis_name="s")

    @functools.partial(
        pl.kernel,
        mesh=mesh,
        out_type=[
            jax.ShapeDtypeStruct((BP, D), jnp.float32),
            jax.ShapeDtypeStruct((BP, D), jnp.float32),
        ],
        scratch_types=[
            pltpu.VMEM((BPW,), jnp.int32),
            pltpu.VMEM((BPW, D), jnp.float32),
            pltpu.VMEM((BPW, D), jnp.float32),
            pltpu.SemaphoreType.DMA,
            pltpu.SemaphoreType.DMA,
        ],
    )
    def _sc_gather(ti_hbm, tv_hbm, idx_hbm, oi_hbm, ov_hbm,
                   idx_v, ri_v, rv_v, s1, s2):
        wid = lax.axis_index("s") * _NC + lax.axis_index("c")
        base = wid * BPW
        pltpu.sync_copy(idx_hbm.at[pl.ds(base, BPW)], idx_v)
        c1 = pltpu.async_copy(ti_hbm.at[idx_v], ri_v, s1)
        c2 = pltpu.async_copy(tv_hbm.at[idx_v], rv_v, s2)
        c1.wait()
        c2.wait()
        pltpu.sync_copy(ri_v, oi_hbm.at[pl.ds(base, BPW)])
        pltpu.sync_copy(rv_v, ov_hbm.at[pl.ds(base, BPW)])

    return _sc_gather


def _attn_block(xi_ref, xv_ref, fvi_ref, fvv_ref, wut_ref, but_ref, wvt_ref,
                wet_ref, sgr_ref, srg_ref, oi_ref, ov_ref):
    xiT = xi_ref[...].T
    xvT = xv_ref[...].T
    wut = wut_ref[...].astype(jnp.bfloat16)
    uiT = jnp.dot(wut, xiT.astype(jnp.bfloat16),
                  preferred_element_type=jnp.float32
                  ).astype(jnp.bfloat16)
    uvT = jnp.dot(wut, xvT.astype(jnp.bfloat16),
                  preferred_element_type=jnp.float32).astype(jnp.bfloat16)
    wvt = wvt_ref[...].astype(jnp.bfloat16)
    but = but_ref[...].astype(jnp.bfloat16)
    qiT = jnp.dot(wvt, fvi_ref[0].T.astype(jnp.bfloat16),
                  preferred_element_type=jnp.float32
                  ).astype(jnp.bfloat16) + but
    qvT = jnp.dot(wvt, fvv_ref[0].T.astype(jnp.bfloat16),
                  preferred_element_type=jnp.float32).astype(jnp.bfloat16) + but

    smat_gr = sgr_ref[...]
    smat_rg = srg_ref[...]
    wet = wet_ref[...].astype(jnp.bfloat16)

    one = jnp.bfloat16(1.0)

    def sig(z):
        return one / (one + jnp.exp(-z))

    bigQi = jnp.dot(qiT, smat_gr,
                    preferred_element_type=jnp.float32).astype(jnp.bfloat16)
    bigQv = jnp.dot(qvT, smat_gr,
                    preferred_element_type=jnp.float32).astype(jnp.bfloat16)
    s_ii = sig(uiT + bigQi)
    s_vi = sig(uvT + bigQi)
    s_iv = sig(uiT + bigQv)
    s_vv = sig(uvT + bigQv)
    e_ii = jnp.dot(wet, s_ii, preferred_element_type=jnp.float32)
    e_vi = jnp.dot(wet, s_vi, preferred_element_type=jnp.float32)
    e_iv = jnp.dot(wet, s_iv, preferred_element_type=jnp.float32)
    e_vv = jnp.dot(wet, s_vv, preferred_element_type=jnp.float32)

    def one_query(ei, ev, out_ref):
        m = jnp.max(jnp.maximum(ei, ev))
        wi = jnp.exp(ei - m)
        wv_ = jnp.exp(ev - m)
        denom = jnp.dot(wi + wv_, smat_rg,
                        preferred_element_type=jnp.float32)
        zT = xiT * wi + xvT * wv_
        rstT = jnp.dot(zT, smat_rg,
                       preferred_element_type=jnp.float32)
        out_ref[...] = (rstT / denom).T[None]

    one_query(e_ii, e_vi, oi_ref)
    one_query(e_iv, e_vv, ov_ref)


def _tc_call(feat_invar, feat_var, fvi, fvv, wut, but, wvt, wet, sgr, srg):
    full = lambda shape: pl.BlockSpec(shape, lambda i: (0, 0))
    return pl.pallas_call(
        _attn_block,
        grid=(GRID,),
        in_specs=[
            pl.BlockSpec((R, D), lambda i: (i, 0)),
            pl.BlockSpec((R, D), lambda i: (i, 0)),
            pl.BlockSpec((1, GB, D), lambda i: (i, 0, 0)),
            pl.BlockSpec((1, GB, D), lambda i: (i, 0, 0)),
            full((H, D)),
            full((H, 1)),
            full((H, D)),
            full((1, H)),
            full((GB, R)),
            full((R, GB)),
        ],
        out_specs=[
            pl.BlockSpec((1, GB, D), lambda i: (i, 0, 0)),
            pl.BlockSpec((1, GB, D), lambda i: (i, 0, 0)),
        ],
        out_shape=[
            jax.ShapeDtypeStruct((GRID, GB, D), jnp.float32),
            jax.ShapeDtypeStruct((GRID, GB, D), jnp.float32),
        ],
    )(feat_invar, feat_var, fvi, fvv, wut, but, wvt, wet, sgr, srg)


def kernel(feat_invar, feat_var, last_nodes, Wu, bu, Wv, We):
    idx = jnp.pad(last_nodes.astype(jnp.int32), (0, BP - B))
    fvi, fvv = _get_sc_gather()(feat_invar, feat_var, idx)
    onehot = (np.arange(R)[:, None] // NPG) == np.arange(GB)[None, :]
    srg = jnp.asarray(onehot, jnp.float32)
    sgr = jnp.asarray(onehot.T, jnp.bfloat16)
    fvi = fvi[:B].reshape(GRID, GB, D)
    fvv = fvv[:B].reshape(GRID, GB, D)
    o3i, o3v = _tc_call(feat_invar, feat_var, fvi, fvv,
                        Wu.T, bu.reshape(H, 1), Wv.T, We.reshape(1, H),
                        sgr, srg)
    rst_i = o3i.reshape(B, D)
    rst_v = o3v.reshape(B, D)
    return rst_i[:, None, :], rst_v[:, None, :]
